# Initial kernel scaffold; baseline (speedup 1.0000x reference)
import jax
import jax.numpy as jnp
from jax.experimental import pallas as pl


def kernel(t, betas):
    raise NotImplementedError("write your pallas kernel here")


# speedup vs baseline: 4.5676x; 4.5676x over previous
import functools

import jax
import jax.numpy as jnp
from jax import lax
from jax.experimental import pallas as pl
from jax.experimental.pallas import tpu as pltpu
from jax.experimental.pallas import tpu_sc as plsc

_LANES = 16


def _gather_body(t_hbm, betas_hbm, out_hbm, tab_v, idx_v, out_v, *,
                 n_workers, b_per_w):
    wid = lax.axis_index("s") * 2 + lax.axis_index("c")
    base = wid * b_per_w
    pltpu.sync_copy(betas_hbm, tab_v)
    pltpu.sync_copy(t_hbm.at[pl.ds(base, b_per_w)], idx_v)
    for i in range(b_per_w // _LANES):
        idx = idx_v[pl.ds(i * _LANES, _LANES)]
        out_v[pl.ds(i * _LANES, _LANES)] = plsc.load_gather(tab_v, [idx])
    pltpu.sync_copy(out_v, out_hbm.at[pl.ds(base, b_per_w)])


def kernel(t, betas):
    b = t.shape[0]
    n_workers = 32
    b_per_w = b // n_workers
    v_pad = 1024
    tab = jnp.zeros((v_pad,), jnp.float32).at[: betas.shape[0]].set(betas)
    t32 = t.astype(jnp.int32)

    mesh = plsc.VectorSubcoreMesh(core_axis_name="c", subcore_axis_name="s")
    run = pl.kernel(
        functools.partial(_gather_body, n_workers=n_workers, b_per_w=b_per_w),
        mesh=mesh,
        compiler_params=pltpu.CompilerParams(needs_layout_passes=False),
        out_type=jax.ShapeDtypeStruct((b,), jnp.float32),
        scratch_types=[
            pltpu.VMEM((v_pad,), jnp.float32),
            pltpu.VMEM((b_per_w,), jnp.int32),
            pltpu.VMEM((b_per_w,), jnp.float32),
        ],
    )
    return run(t32, tab)


# speedup vs baseline: 4.6783x; 1.0242x over previous
import functools

import jax
import jax.numpy as jnp
from jax import lax
from jax.experimental import pallas as pl
from jax.experimental.pallas import tpu as pltpu
from jax.experimental.pallas import tpu_sc as plsc

_LANES = 16


def _gather_body(t_hbm, betas_hbm, out_hbm, tab_v, idx_v, out_v, sem_t,
                 sem_i, *, b_per_w):
    wid = lax.axis_index("s") * 2 + lax.axis_index("c")
    base = wid * b_per_w
    cp_t = pltpu.async_copy(betas_hbm, tab_v, sem_t)
    cp_i = pltpu.async_copy(t_hbm.at[pl.ds(base, b_per_w)], idx_v, sem_i)
    cp_t.wait()
    cp_i.wait()
    for i in range(b_per_w // _LANES):
        idx = idx_v[pl.ds(i * _LANES, _LANES)]
        out_v[pl.ds(i * _LANES, _LANES)] = plsc.load_gather(tab_v, [idx])
    pltpu.sync_copy(out_v, out_hbm.at[pl.ds(base, b_per_w)])


def kernel(t, betas):
    b = t.shape[0]
    n_workers = 32
    b_per_w = b // n_workers
    t32 = t.astype(jnp.int32)
    v = betas.shape[0]

    mesh = plsc.VectorSubcoreMesh(core_axis_name="c", subcore_axis_name="s")
    run = pl.kernel(
        functools.partial(_gather_body, b_per_w=b_per_w),
        mesh=mesh,
        compiler_params=pltpu.CompilerParams(needs_layout_passes=False),
        out_type=jax.ShapeDtypeStruct((b,), jnp.float32),
        scratch_types=[
            pltpu.VMEM((v,), jnp.float32),
            pltpu.VMEM((b_per_w,), jnp.int32),
            pltpu.VMEM((b_per_w,), jnp.float32),
            pltpu.SemaphoreType.DMA,
            pltpu.SemaphoreType.DMA,
        ],
    )
    return run(t32, betas)
